# baseline (speedup 1.0000x reference)
import functools

import jax
import jax.numpy as jnp
from jax import lax
from jax.experimental import pallas as pl
from jax.experimental.pallas import tpu as pltpu
from jax.experimental.pallas import tpu_sc as plsc

B = 1024
SEQ = 200
CH = 32
GRAM = 3
OUT_N = SEQ + GRAM - 1
QCH = CH // 4
NQ = 4 * SEQ

_info = plsc.get_sparse_core_info()
_NC = _info.num_cores
_NS = _info.num_subcores
_NW = _NC * _NS

_QPW = NQ // _NW
_NB = 6
_PD = 2
_NZSTRIP = 2 * GRAM * 4
_ZW0 = _NW - _NZSTRIP


def _copy_quarters(xt, yt, bufs, rsems, wsems, q_start, nq):
    reads = [None] * nq
    writes = [[] for _ in range(nq)]
    waited = set()

    def issue_read(k):
        g = q_start + k
        return pltpu.async_copy(
            xt.at[pl.ds(g // 4, 1), pl.ds((g % 4) * QCH, QCH)],
            bufs[k % _NB], rsems.at[k % _NB])

    for p in range(min(_PD, nq)):
        reads[p] = issue_read(p)
    for k in range(nq):
        reads[k].wait()
        g = q_start + k
        t, co = g // 4, (g % 4) * QCH
        for j in range(GRAM):
            writes[k].append(pltpu.async_copy(
                bufs[k % _NB],
                yt.at[j, pl.ds(t + (GRAM - 1 - j), 1), pl.ds(co, QCH)],
                wsems.at[k % _NB, j]))
        nxt = k + _PD
        if nxt < nq:
            old = k - (_NB - _PD)
            if old >= 0:
                for h in writes[old]:
                    h.wait()
                waited.add(old)
            reads[nxt] = issue_read(nxt)
    for k in range(nq):
        if k not in waited:
            for h in writes[k]:
                h.wait()


def _body(xt, yt, b0, b1, b2, b3, b4, b5, zero_v, rsems, wsems, zsem):
    wid = lax.axis_index("s") * _NC + lax.axis_index("c")
    bufs = (b0, b1, b2, b3, b4, b5)

    _copy_quarters(xt, yt, bufs, rsems, wsems, _QPW * wid, _QPW)

    @pl.when(wid >= _ZW0)
    def _():
        z16 = jnp.zeros((16,), jnp.float32)

        def zinit(i, carry):
            r = i // (B // 16)
            k = (i % (B // 16)) * 16
            zero_v[0, r, pl.ds(k, 16)] = z16
            return carry

        lax.fori_loop(0, QCH * (B // 16), zinit, 0)
        z = wid - _ZW0
        blk = z // 4
        co = (z % 4) * QCH
        jz = blk // 2
        iz = jnp.where(blk % 2 == 0, SEQ * (blk // 4),
                       1 + SEQ * jnp.int32(blk >= GRAM))
        pltpu.async_copy(zero_v, yt.at[jz, pl.ds(iz, 1), pl.ds(co, QCH)],
                         zsem).wait()


_ngram_sc = functools.partial(
    pl.kernel,
    out_type=jax.ShapeDtypeStruct((GRAM, OUT_N, CH, B), jnp.float32),
    mesh=plsc.VectorSubcoreMesh(core_axis_name="c", subcore_axis_name="s"),
    scratch_types=(
        [pltpu.VMEM((1, QCH, B), jnp.float32) for _ in range(_NB)]
        + [
            pltpu.VMEM((1, QCH, B), jnp.float32),
            pltpu.SemaphoreType.DMA((_NB,)),
            pltpu.SemaphoreType.DMA((_NB, GRAM)),
            pltpu.SemaphoreType.DMA,
        ]
    ),
)(_body)


def kernel(inputs):
    xt = jnp.transpose(inputs, (1, 2, 0))
    yt = _ngram_sc(xt)
    return jnp.transpose(yt, (3, 0, 1, 2))


# speedup vs baseline: 1.0637x; 1.0637x over previous
import functools

import jax
import jax.numpy as jnp
from jax import lax
from jax.experimental import pallas as pl
from jax.experimental.pallas import tpu as pltpu
from jax.experimental.pallas import tpu_sc as plsc

B = 1024
SEQ = 200
CH = 32
GRAM = 3
OUT_N = SEQ + GRAM - 1
HCH = CH // 2
NHALF = 2 * SEQ

_info = plsc.get_sparse_core_info()
_NC = _info.num_cores
_NS = _info.num_subcores
_NW = _NC * _NS

_N13 = NHALF - 12 * _NW
_NB = 6
_PD = 2
_ZW0 = _NW - 2 * GRAM


def _copy_halves(xt, yt, bufs, rsems, wsems, h_start, nh):
    reads = [None] * nh
    writes = [[] for _ in range(nh)]
    waited = set()

    def issue_read(h):
        g = h_start + h
        return pltpu.async_copy(
            xt.at[pl.ds(g // 2, 1), pl.ds((g % 2) * HCH, HCH)],
            bufs[h % _NB], rsems.at[h % _NB])

    for p in range(min(_PD, nh)):
        reads[p] = issue_read(p)
    for k in range(nh):
        reads[k].wait()
        g = h_start + k
        t, co = g // 2, (g % 2) * HCH
        for j in range(GRAM):
            writes[k].append(pltpu.async_copy(
                bufs[k % _NB],
                yt.at[j, pl.ds(t + (GRAM - 1 - j), 1), pl.ds(co, HCH)],
                wsems.at[k % _NB, j]))
        nxt = k + _PD
        if nxt < nh:
            old = k - (_NB - _PD)
            if old >= 0:
                for h in writes[old]:
                    h.wait()
                waited.add(old)
            reads[nxt] = issue_read(nxt)
    for k in range(nh):
        if k not in waited:
            for h in writes[k]:
                h.wait()


def _body(xt, yt, b0, b1, b2, b3, b4, b5, zero_v, rsems, wsems, zsems):
    wid = lax.axis_index("s") * _NC + lax.axis_index("c")
    bufs = (b0, b1, b2, b3, b4, b5)

    @pl.when(wid < _N13)
    def _():
        _copy_halves(xt, yt, bufs, rsems, wsems, 13 * wid, 13)

    @pl.when(wid >= _N13)
    def _():
        _copy_halves(xt, yt, bufs, rsems, wsems, 12 * wid + _N13, 12)

    @pl.when(wid >= _ZW0)
    def _():
        z16 = jnp.zeros((16,), jnp.float32)

        def zinit(i, carry):
            r = i // (B // 16)
            k = (i % (B // 16)) * 16
            zero_v[0, r, pl.ds(k, 16)] = z16
            return carry

        lax.fori_loop(0, 8 * (B // 16), zinit, 0)
        z = wid - _ZW0
        jz = z // 2
        iz = jnp.where(z % 2 == 0, SEQ * (z // 4),
                       1 + SEQ * jnp.int32(z >= GRAM))
        zw = [pltpu.async_copy(zero_v,
                               yt.at[jz, pl.ds(iz, 1), pl.ds(c, 8)],
                               zsems.at[c // 8])
              for c in range(0, CH, 8)]
        for h in zw:
            h.wait()


_ngram_sc = functools.partial(
    pl.kernel,
    out_type=jax.ShapeDtypeStruct((GRAM, OUT_N, CH, B), jnp.float32),
    mesh=plsc.VectorSubcoreMesh(core_axis_name="c", subcore_axis_name="s"),
    scratch_types=(
        [pltpu.VMEM((1, HCH, B), jnp.float32) for _ in range(_NB)]
        + [
            pltpu.VMEM((1, 8, B), jnp.float32),
            pltpu.SemaphoreType.DMA((_NB,)),
            pltpu.SemaphoreType.DMA((_NB, GRAM)),
            pltpu.SemaphoreType.DMA((CH // 8,)),
        ]
    ),
)(_body)


def kernel(inputs):
    xt = jnp.transpose(inputs, (1, 2, 0))
    yt = _ngram_sc(xt)
    return jnp.transpose(yt, (3, 0, 1, 2))
